# Initial kernel scaffold; baseline (speedup 1.0000x reference)
import functools

import jax
import jax.numpy as jnp
from jax import lax
from jax.experimental import pallas as pl
from jax.experimental.pallas import tpu as pltpu
from jax.experimental.pallas import tpu_sc as plsc

B = 64
N = 32768
C = 16
K = 4

NW = 32
BPW = B // NW
CH = 2048
NCHUNK = N // CH
BLK = 16
NBLK = CH // BLK


def _tree_reduce(vals, op):
    while len(vals) > 1:
        nxt = [op(vals[i], vals[i + 1]) for i in range(0, len(vals) - 1, 2)]
        if len(vals) % 2:
            nxt.append(vals[-1])
        vals = nxt
    return vals[0]


def _insert_top(t, v):
    out = []
    cur = v
    for i in range(3):
        out.append(jnp.maximum(t[i], cur))
        cur = jnp.minimum(t[i], cur)
    out.append(jnp.maximum(t[3], cur))
    return out


def _insert_bot(bt, v):
    out = []
    cur = v
    for i in range(3):
        out.append(jnp.minimum(bt[i], cur))
        cur = jnp.maximum(bt[i], cur)
    out.append(jnp.minimum(bt[3], cur))
    return out


def _process_chunk(buf, state):
    @pl.loop(0, NBLK)
    def _blk_loop(blk):
        base = blk * BLK
        vs = [buf[base + i, :] for i in range(BLK)]
        bm = _tree_reduce(list(vs), jnp.maximum)
        bn = _tree_reduce(list(vs), jnp.minimum)
        t4 = state[3, :]
        b4 = state[7, :]
        need = jnp.any(bm > t4) | jnp.any(bn < b4)

        @pl.when(need)
        def _rescan():
            t = [state[i, :] for i in range(4)]
            bt = [state[4 + i, :] for i in range(4)]
            for v in vs:
                t = _insert_top(t, v)
                bt = _insert_bot(bt, v)
            for i in range(4):
                state[i, :] = t[i]
                state[4 + i, :] = bt[i]


def _milpool_sc(x):
    mesh = plsc.VectorSubcoreMesh(core_axis_name="c", subcore_axis_name="s")

    @functools.partial(
        pl.kernel,
        out_type=jax.ShapeDtypeStruct((B * 2 * K, C), jnp.float32),
        mesh=mesh,
        scratch_types=[
            pltpu.VMEM((CH, C), jnp.float32),
            pltpu.VMEM((CH, C), jnp.float32),
            pltpu.VMEM((2 * K, C), jnp.float32),
            pltpu.SemaphoreType.DMA,
            pltpu.SemaphoreType.DMA,
        ],
    )
    def k(x_hbm, o_hbm, buf_a, buf_b, state, sem_a, sem_b):
        wid = lax.axis_index("s") * 2 + lax.axis_index("c")
        for bi in range(BPW):
            b = wid * BPW + bi
            row0 = b * N
            for i in range(4):
                state[i, :] = jnp.full((C,), -jnp.inf, dtype=jnp.float32)
                state[4 + i, :] = jnp.full((C,), jnp.inf, dtype=jnp.float32)
            pltpu.async_copy(x_hbm.at[pl.ds(row0, CH), :], buf_a, sem_a)
            pltpu.async_copy(x_hbm.at[pl.ds(row0 + CH, CH), :], buf_b, sem_b)

            @pl.loop(0, NCHUNK, step=2)
            def _chunk_loop(j):
                pltpu.make_async_copy(
                    x_hbm.at[pl.ds(row0 + j * CH, CH), :], buf_a, sem_a
                ).wait()
                _process_chunk(buf_a, state)

                @pl.when(j + 2 < NCHUNK)
                def _refill_a():
                    pltpu.async_copy(
                        x_hbm.at[pl.ds(row0 + (j + 2) * CH, CH), :], buf_a, sem_a
                    )

                pltpu.make_async_copy(
                    x_hbm.at[pl.ds(row0 + (j + 1) * CH, CH), :], buf_b, sem_b
                ).wait()
                _process_chunk(buf_b, state)

                @pl.when(j + 3 < NCHUNK)
                def _refill_b():
                    pltpu.async_copy(
                        x_hbm.at[pl.ds(row0 + (j + 3) * CH, CH), :], buf_b, sem_b
                    )

            pltpu.sync_copy(state, o_hbm.at[pl.ds(b * 2 * K, 2 * K), :])

    return k(x)


@jax.jit
def kernel(inputs):
    x = inputs.reshape(B * N, C)
    out = _milpool_sc(x)
    return out.reshape(B, 2 * K, C)


# speedup vs baseline: 39.3617x; 39.3617x over previous
import dataclasses
import functools

import jax
import jax.numpy as jnp
from jax import lax
from jax.experimental import pallas as pl
from jax.experimental.pallas import tpu as pltpu
from jax.experimental.pallas import tpu_sc as plsc

B = 64
N = 32768
C = 16
K = 4

NW = 32
BPW = B // NW
CH = 2048
NCHUNK = N // CH
BLK = 16
NBLK = CH // BLK


def _tree_reduce(vals, op):
    while len(vals) > 1:
        nxt = [op(vals[i], vals[i + 1]) for i in range(0, len(vals) - 1, 2)]
        if len(vals) % 2:
            nxt.append(vals[-1])
        vals = nxt
    return vals[0]


def _insert_top(t, v):
    out = []
    cur = v
    for i in range(3):
        out.append(jnp.maximum(t[i], cur))
        cur = jnp.minimum(t[i], cur)
    out.append(jnp.maximum(t[3], cur))
    return out


def _insert_bot(bt, v):
    out = []
    cur = v
    for i in range(3):
        out.append(jnp.minimum(bt[i], cur))
        cur = jnp.maximum(bt[i], cur)
    out.append(jnp.minimum(bt[3], cur))
    return out


def _process_chunk(buf, state):
    @pl.loop(0, NBLK)
    def _blk_loop(blk):
        base = blk * BLK
        vs = [buf[base + i, :] for i in range(BLK)]
        bm = _tree_reduce(list(vs), jnp.maximum)
        bn = _tree_reduce(list(vs), jnp.minimum)
        t4 = state[3, :]
        b4 = state[7, :]
        need = jnp.any(bm > t4) | jnp.any(bn < b4)

        @pl.when(need)
        def _rescan():
            t = [state[i, :] for i in range(4)]
            bt = [state[4 + i, :] for i in range(4)]
            for v in vs:
                t = _insert_top(t, v)
                bt = _insert_bot(bt, v)
            for i in range(4):
                state[i, :] = t[i]
                state[4 + i, :] = bt[i]


def _milpool_sc(x):
    mesh = plsc.VectorSubcoreMesh(core_axis_name="c", subcore_axis_name="s")
    cp = pltpu.CompilerParams()
    for fld, val in (("needs_layout_passes", False),
                     ("use_tc_tiling_on_sc", False)):
        if fld in pltpu.CompilerParams.__dataclass_fields__:
            cp = dataclasses.replace(cp, **{fld: val})

    @functools.partial(
        pl.kernel,
        compiler_params=cp,
        out_type=jax.ShapeDtypeStruct((B * 2 * K, C), jnp.float32),
        mesh=mesh,
        scratch_types=[
            pltpu.VMEM((CH, C), jnp.float32),
            pltpu.VMEM((CH, C), jnp.float32),
            pltpu.VMEM((2 * K, C), jnp.float32),
            pltpu.SemaphoreType.DMA,
            pltpu.SemaphoreType.DMA,
        ],
    )
    def k(x_hbm, o_hbm, buf_a, buf_b, state, sem_a, sem_b):
        wid = lax.axis_index("s") * 2 + lax.axis_index("c")
        for bi in range(BPW):
            b = wid * BPW + bi
            row0 = b * N
            for i in range(4):
                state[i, :] = jnp.full((C,), -jnp.inf, dtype=jnp.float32)
                state[4 + i, :] = jnp.full((C,), jnp.inf, dtype=jnp.float32)
            pltpu.async_copy(x_hbm.at[pl.ds(row0, CH), :], buf_a, sem_a)
            pltpu.async_copy(x_hbm.at[pl.ds(row0 + CH, CH), :], buf_b, sem_b)

            @pl.loop(0, NCHUNK, step=2)
            def _chunk_loop(j):
                pltpu.make_async_copy(
                    x_hbm.at[pl.ds(row0 + j * CH, CH), :], buf_a, sem_a
                ).wait()
                _process_chunk(buf_a, state)

                @pl.when(j + 2 < NCHUNK)
                def _refill_a():
                    pltpu.async_copy(
                        x_hbm.at[pl.ds(row0 + (j + 2) * CH, CH), :], buf_a, sem_a
                    )

                pltpu.make_async_copy(
                    x_hbm.at[pl.ds(row0 + (j + 1) * CH, CH), :], buf_b, sem_b
                ).wait()
                _process_chunk(buf_b, state)

                @pl.when(j + 3 < NCHUNK)
                def _refill_b():
                    pltpu.async_copy(
                        x_hbm.at[pl.ds(row0 + (j + 3) * CH, CH), :], buf_b, sem_b
                    )

            pltpu.sync_copy(state, o_hbm.at[pl.ds(b * 2 * K, 2 * K), :])

    return k(x)


@jax.jit
def kernel(inputs):
    x = inputs.reshape(B * N, C)
    out = _milpool_sc(x)
    return out.reshape(B, 2 * K, C)


# speedup vs baseline: 295.7547x; 7.5138x over previous
import dataclasses
import functools

import jax
import jax.numpy as jnp
from jax import lax
from jax.experimental import pallas as pl
from jax.experimental.pallas import tpu as pltpu
from jax.experimental.pallas import tpu_sc as plsc

B = 64
N = 32768
C = 16
K = 4

NW = 32
NSS = B * C
SPW = NSS // NW
NT = 256
ROWS = NT
BLKV = 16
NBLK = (NT * 128) // (BLKV * 16)
NEG = float("-inf")
POS = float("inf")


def _tree_reduce(vals, op):
    while len(vals) > 1:
        nxt = [op(vals[i], vals[i + 1]) for i in range(0, len(vals) - 1, 2)]
        if len(vals) % 2:
            nxt.append(vals[-1])
        vals = nxt
    return vals[0]


def _insert_top(t, v):
    out = []
    cur = v
    for i in range(3):
        out.append(jnp.maximum(t[i], cur))
        cur = jnp.minimum(t[i], cur)
    out.append(jnp.maximum(t[3], cur))
    return out


def _insert_bot(bt, v):
    out = []
    cur = v
    for i in range(3):
        out.append(jnp.minimum(bt[i], cur))
        cur = jnp.maximum(bt[i], cur)
    out.append(jnp.minimum(bt[3], cur))
    return out


def _sortd(v):
    return plsc.sort_key_val(v, v, descending=True)[0]


def _sorta(v):
    return plsc.sort_key_val(v, v, descending=False)[0]


def _milpool_sc(x5):
    mesh = plsc.VectorSubcoreMesh(core_axis_name="c", subcore_axis_name="s")
    cp = pltpu.CompilerParams()
    for fld, val in (("needs_layout_passes", False),
                     ("use_tc_tiling_on_sc", False)):
        if fld in pltpu.CompilerParams.__dataclass_fields__:
            cp = dataclasses.replace(cp, **{fld: val})

    @functools.partial(
        pl.kernel,
        compiler_params=cp,
        out_type=jax.ShapeDtypeStruct((NSS, 2, 16), jnp.float32),
        mesh=mesh,
        scratch_types=[
            pltpu.VMEM((ROWS, 128), jnp.float32),
            pltpu.VMEM((ROWS, 128), jnp.float32),
            pltpu.VMEM((NBLK, 16), jnp.float32),
            pltpu.VMEM((NBLK, 16), jnp.float32),
            pltpu.VMEM((2, 16), jnp.float32),
            pltpu.SemaphoreType.DMA,
            pltpu.SemaphoreType.DMA,
        ],
    )
    def k(x_hbm, o_hbm, buf_a, buf_b, bm_ref, bn_ref, ost, sem_a, sem_b):
        wid = lax.axis_index("s") * 2 + lax.axis_index("c")
        ss0 = wid * SPW
        iota = lax.iota(jnp.int32, 16)
        lt4 = iota < 4
        cols = [iota + 16 * q for q in range(8)]

        def src(ss):
            b = ss // 16
            ct = (ss // 8) % 2
            cr = ss % 8
            return x_hbm.at[b, ct, pl.ds(0, NT), cr, :]

        def process(buf, ss):
            @pl.loop(0, NBLK)
            def _p1(blk):
                r0 = blk * 2
                vs = [buf[r0 + (q // 8), pl.ds(16 * (q % 8), 16)]
                      for q in range(BLKV)]
                bm_ref[blk, :] = _tree_reduce(list(vs), jnp.maximum)
                bn_ref[blk, :] = _tree_reduce(list(vs), jnp.minimum)

            def p2_body_top(j, carry):
                t1, t2, t3, t4, i1, i2, i3, i4 = carry
                cur = bm_ref[j, :]
                cid = jnp.full((16,), 0, jnp.int32) + j
                for _ in range(1):
                    c = cur > t1
                    t1, cur = jnp.where(c, cur, t1), jnp.where(c, t1, cur)
                    i1, cid = jnp.where(c, cid, i1), jnp.where(c, i1, cid)
                    c = cur > t2
                    t2, cur = jnp.where(c, cur, t2), jnp.where(c, t2, cur)
                    i2, cid = jnp.where(c, cid, i2), jnp.where(c, i2, cid)
                    c = cur > t3
                    t3, cur = jnp.where(c, cur, t3), jnp.where(c, t3, cur)
                    i3, cid = jnp.where(c, cid, i3), jnp.where(c, i3, cid)
                    c = cur > t4
                    t4 = jnp.where(c, cur, t4)
                    i4 = jnp.where(c, cid, i4)
                return t1, t2, t3, t4, i1, i2, i3, i4

            def p2_body_bot(j, carry):
                t1, t2, t3, t4, i1, i2, i3, i4 = carry
                cur = bn_ref[j, :]
                cid = jnp.full((16,), 0, jnp.int32) + j
                for _ in range(1):
                    c = cur < t1
                    t1, cur = jnp.where(c, cur, t1), jnp.where(c, t1, cur)
                    i1, cid = jnp.where(c, cid, i1), jnp.where(c, i1, cid)
                    c = cur < t2
                    t2, cur = jnp.where(c, cur, t2), jnp.where(c, t2, cur)
                    i2, cid = jnp.where(c, cid, i2), jnp.where(c, i2, cid)
                    c = cur < t3
                    t3, cur = jnp.where(c, cur, t3), jnp.where(c, t3, cur)
                    i3, cid = jnp.where(c, cid, i3), jnp.where(c, i3, cid)
                    c = cur < t4
                    t4 = jnp.where(c, cur, t4)
                    i4 = jnp.where(c, cid, i4)
                return t1, t2, t3, t4, i1, i2, i3, i4

            ninf = jnp.full((16,), NEG, jnp.float32)
            pinf = jnp.full((16,), POS, jnp.float32)
            zid = jnp.full((16,), 0, jnp.int32)
            tcar = lax.fori_loop(
                0, NBLK, p2_body_top,
                (ninf, ninf, ninf, ninf, zid, zid, zid, zid))
            bcar = lax.fori_loop(
                0, NBLK, p2_body_bot,
                (pinf, pinf, pinf, pinf, zid, zid, zid, zid))
            top_ids = tcar[4:]
            bot_ids = bcar[4:]

            t = [ninf, ninf, ninf, ninf]
            for r in range(4):
                row0 = top_ids[r] * 2
                for q in range(BLKV):
                    rowv = row0 + (q // 8)
                    v = plsc.load_gather(buf, [rowv, cols[q % 8]])
                    t = _insert_top(t, v)
            bt = [pinf, pinf, pinf, pinf]
            for r in range(4):
                row0 = bot_ids[r] * 2
                for q in range(BLKV):
                    rowv = row0 + (q // 8)
                    v = plsc.load_gather(buf, [rowv, cols[q % 8]])
                    bt = _insert_bot(bt, v)

            s = _sortd(t[0])
            for v in t[1:]:
                s = _sortd(jnp.where(lt4, s, jnp.flip(_sortd(v))))
            sb = _sorta(bt[0])
            for v in bt[1:]:
                sb = _sorta(jnp.where(lt4, sb, jnp.flip(_sorta(v))))

            ost[0, :] = s
            ost[1, :] = sb
            pltpu.sync_copy(ost, o_hbm.at[ss])

        pltpu.async_copy(src(ss0), buf_a, sem_a)
        pltpu.async_copy(src(ss0 + 1), buf_b, sem_b)

        @pl.loop(0, SPW // 2)
        def _pair(p):
            ssa = ss0 + 2 * p
            pltpu.make_async_copy(src(ssa), buf_a, sem_a).wait()
            process(buf_a, ssa)

            @pl.when(2 * p + 2 < SPW)
            def _refill_a():
                pltpu.async_copy(src(ssa + 2), buf_a, sem_a)

            pltpu.make_async_copy(src(ssa + 1), buf_b, sem_b).wait()
            process(buf_b, ssa + 1)

            @pl.when(2 * p + 3 < SPW)
            def _refill_b():
                pltpu.async_copy(src(ssa + 3), buf_b, sem_b)

    return k(x5)


@jax.jit
def kernel(inputs):
    x5 = inputs.reshape(B, NT, 128, 2, 8).transpose(0, 3, 1, 4, 2)
    o = _milpool_sc(x5)
    o6 = o.reshape(B, 2, 8, 2, 16)[:, :, :, :, :K]
    return jnp.transpose(o6, (0, 3, 4, 1, 2)).reshape(B, 2 * K, C)


# speedup vs baseline: 385.1813x; 1.3024x over previous
import dataclasses
import functools

import jax
import jax.numpy as jnp
from jax import lax
from jax.experimental import pallas as pl
from jax.experimental.pallas import tpu as pltpu
from jax.experimental.pallas import tpu_sc as plsc

B = 64
N = 32768
C = 16
K = 4

NW = 32
NSS = B * C
SPW = NSS // NW
NT = 256
ROWS = NT
BLKV = 32
RPB = BLKV // 8
NBLK = (NT * 128) // (BLKV * 16)
NEG = float("-inf")
POS = float("inf")


def _tree_reduce(vals, op):
    while len(vals) > 1:
        nxt = [op(vals[i], vals[i + 1]) for i in range(0, len(vals) - 1, 2)]
        if len(vals) % 2:
            nxt.append(vals[-1])
        vals = nxt
    return vals[0]


def _insert_top(t, v):
    out = []
    cur = v
    for i in range(3):
        out.append(jnp.maximum(t[i], cur))
        cur = jnp.minimum(t[i], cur)
    out.append(jnp.maximum(t[3], cur))
    return out


def _insert_bot(bt, v):
    out = []
    cur = v
    for i in range(3):
        out.append(jnp.minimum(bt[i], cur))
        cur = jnp.maximum(bt[i], cur)
    out.append(jnp.minimum(bt[3], cur))
    return out


def _ladder_ids(vals, ids, cur, cid, gt):
    o_v, o_i = [], []
    for r in range(3):
        c = gt(cur, vals[r])
        o_v.append(jnp.where(c, cur, vals[r]))
        o_i.append(jnp.where(c, cid, ids[r]))
        cur, cid = jnp.where(c, vals[r], cur), jnp.where(c, ids[r], cid)
    c = gt(cur, vals[3])
    o_v.append(jnp.where(c, cur, vals[3]))
    o_i.append(jnp.where(c, cid, ids[3]))
    return o_v, o_i


def _sortd(v):
    return plsc.sort_key_val(v, v, descending=True)[0]


def _sorta(v):
    return plsc.sort_key_val(v, v, descending=False)[0]


def _milpool_sc(x5):
    mesh = plsc.VectorSubcoreMesh(core_axis_name="c", subcore_axis_name="s")
    cp = pltpu.CompilerParams()
    for fld, val in (("needs_layout_passes", False),
                     ("use_tc_tiling_on_sc", False)):
        if fld in pltpu.CompilerParams.__dataclass_fields__:
            cp = dataclasses.replace(cp, **{fld: val})

    @functools.partial(
        pl.kernel,
        compiler_params=cp,
        out_type=jax.ShapeDtypeStruct((NSS, 2, 16), jnp.float32),
        mesh=mesh,
        scratch_types=[
            pltpu.VMEM((ROWS, 128), jnp.float32),
            pltpu.VMEM((ROWS, 128), jnp.float32),
            pltpu.VMEM((2, 16), jnp.float32),
            pltpu.SemaphoreType.DMA,
            pltpu.SemaphoreType.DMA,
        ],
    )
    def k(x_hbm, o_hbm, buf_a, buf_b, ost, sem_a, sem_b):
        wid = lax.axis_index("s") * 2 + lax.axis_index("c")
        ss0 = wid * SPW
        iota = lax.iota(jnp.int32, 16)
        lt4 = iota < 4
        cols = [iota + 16 * q for q in range(8)]
        ninf = jnp.full((16,), NEG, jnp.float32)
        pinf = jnp.full((16,), POS, jnp.float32)
        zid = jnp.full((16,), 0, jnp.int32)

        def src(ss):
            b = ss // 16
            ct = (ss // 8) % 2
            cr = ss % 8
            return x_hbm.at[b, ct, pl.ds(0, NT), cr, :]

        def process(buf, ss):
            def body(blk, carry):
                tv = list(carry[0:4])
                ti = list(carry[4:8])
                bv = list(carry[8:12])
                bi = list(carry[12:16])
                r0 = blk * RPB
                vs = [buf[r0 + (q // 8), pl.ds(16 * (q % 8), 16)]
                      for q in range(BLKV)]
                bm = _tree_reduce(list(vs), jnp.maximum)
                bn = _tree_reduce(list(vs), jnp.minimum)
                cid = zid + blk
                tv, ti = _ladder_ids(tv, ti, bm, cid, lambda a, b: a > b)
                bv, bi = _ladder_ids(bv, bi, bn, cid, lambda a, b: a < b)
                return tuple(tv) + tuple(ti) + tuple(bv) + tuple(bi)

            carry = lax.fori_loop(
                0, NBLK, body,
                (ninf, ninf, ninf, ninf, zid, zid, zid, zid,
                 pinf, pinf, pinf, pinf, zid, zid, zid, zid))
            top_ids = carry[4:8]
            bot_ids = carry[12:16]

            t = [ninf, ninf, ninf, ninf]
            for r in range(4):
                row0 = top_ids[r] * RPB
                for q in range(BLKV):
                    rowv = row0 + (q // 8)
                    v = plsc.load_gather(buf, [rowv, cols[q % 8]])
                    t = _insert_top(t, v)
            bt = [pinf, pinf, pinf, pinf]
            for r in range(4):
                row0 = bot_ids[r] * RPB
                for q in range(BLKV):
                    rowv = row0 + (q // 8)
                    v = plsc.load_gather(buf, [rowv, cols[q % 8]])
                    bt = _insert_bot(bt, v)

            s = _sortd(t[0])
            for v in t[1:]:
                s = _sortd(jnp.where(lt4, s, jnp.flip(_sortd(v))))
            sb = _sorta(bt[0])
            for v in bt[1:]:
                sb = _sorta(jnp.where(lt4, sb, jnp.flip(_sorta(v))))

            ost[0, :] = s
            ost[1, :] = sb
            pltpu.sync_copy(ost, o_hbm.at[ss])

        pltpu.async_copy(src(ss0), buf_a, sem_a)
        pltpu.async_copy(src(ss0 + 1), buf_b, sem_b)

        @pl.loop(0, SPW // 2)
        def _pair(p):
            ssa = ss0 + 2 * p
            pltpu.make_async_copy(src(ssa), buf_a, sem_a).wait()
            process(buf_a, ssa)

            @pl.when(2 * p + 2 < SPW)
            def _refill_a():
                pltpu.async_copy(src(ssa + 2), buf_a, sem_a)

            pltpu.make_async_copy(src(ssa + 1), buf_b, sem_b).wait()
            process(buf_b, ssa + 1)

            @pl.when(2 * p + 3 < SPW)
            def _refill_b():
                pltpu.async_copy(src(ssa + 3), buf_b, sem_b)

    return k(x5)


@jax.jit
def kernel(inputs):
    x5 = inputs.reshape(B, NT, 128, 2, 8).transpose(0, 3, 1, 4, 2)
    o = _milpool_sc(x5)
    o6 = o.reshape(B, 2, 8, 2, 16)[:, :, :, :, :K]
    return jnp.transpose(o6, (0, 3, 4, 1, 2)).reshape(B, 2 * K, C)
